# baseline (speedup 1.0000x reference)
import dataclasses
import functools

import jax
import jax.numpy as jnp
from jax import lax
from jax.experimental import pallas as pl
from jax.experimental.pallas import tpu as pltpu
from jax.experimental.pallas import tpu_sc as plsc

_NC = 2
_NS = 16
_CH = 128
_NBUF = 3


def _linear_relu(x, W, b):
    n, d = x.shape
    blk = 1000

    def body(x_ref, w_ref, b_ref, o_ref):
        h = jnp.dot(x_ref[...], w_ref[...],
                    preferred_element_type=jnp.float32,
                    precision=jax.lax.Precision.HIGHEST)
        o_ref[...] = jnp.maximum(h + b_ref[...], 0.0)

    return pl.pallas_call(
        body,
        grid=(n // blk,),
        in_specs=[
            pl.BlockSpec((blk, d), lambda i: (i, 0)),
            pl.BlockSpec((d, d), lambda i: (0, 0)),
            pl.BlockSpec((1, d), lambda i: (0, 0)),
        ],
        out_specs=pl.BlockSpec((blk, d), lambda i: (i, 0)),
        out_shape=jax.ShapeDtypeStruct((n, d), jnp.float32),
    )(x, W, b.reshape(1, d))


def _combine_ln(h, a0, a1, g, bt):
    n, d = h.shape
    blk = 1000

    def body(h_ref, a0_ref, a1_ref, g_ref, bt_ref, o_ref):
        s = h_ref[...] + a0_ref[...] + a1_ref[...]
        mu = jnp.mean(s, axis=-1, keepdims=True)
        var = jnp.mean((s - mu) ** 2, axis=-1, keepdims=True)
        o_ref[...] = (s - mu) * jax.lax.rsqrt(var + 1e-5) * g_ref[...] + bt_ref[...]

    return pl.pallas_call(
        body,
        grid=(n // blk,),
        in_specs=[
            pl.BlockSpec((blk, d), lambda i: (i, 0)),
            pl.BlockSpec((blk, d), lambda i: (i, 0)),
            pl.BlockSpec((blk, d), lambda i: (i, 0)),
            pl.BlockSpec((1, d), lambda i: (0, 0)),
            pl.BlockSpec((1, d), lambda i: (0, 0)),
        ],
        out_specs=pl.BlockSpec((blk, d), lambda i: (i, 0)),
        out_shape=jax.ShapeDtypeStruct((n, d), jnp.float32),
    )(h, a0, a1, g.reshape(1, d), bt.reshape(1, d))


def _edge_pass(h, src_flat, dst2d, w_flat, zeros):
    n, d = h.shape
    nw = _NC * _NS
    per_w = src_flat.shape[0] // nw
    n_chunks = per_w // _CH
    assert n_chunks % 4 == 0 and n_chunks >= 8
    slab = ((n + _NS - 1) // _NS + 7) // 8 * 8
    last_slab = n - (_NS - 1) * slab
    assert last_slab > 0 and last_slab % 8 == 0

    mesh = plsc.VectorSubcoreMesh(core_axis_name="c", subcore_axis_name="s")

    cp = pltpu.CompilerParams()
    if "needs_layout_passes" in pltpu.CompilerParams.__dataclass_fields__:
        cp = dataclasses.replace(cp, needs_layout_passes=False)

    @functools.partial(
        pl.kernel,
        compiler_params=cp,
        out_type=jax.ShapeDtypeStruct((_NC, n, d), jnp.float32),
        mesh=mesh,
        scratch_types=[
            pltpu.VMEM((n_chunks, _CH), jnp.int32),
            pltpu.VMEM((4, _CH), jnp.int32),
            pltpu.VMEM((4 * _CH,), jnp.float32),
            pltpu.VMEM((2, _CH, d), jnp.float32),
            pltpu.SemaphoreType.DMA,
            pltpu.SemaphoreType.DMA,
            pltpu.SemaphoreType.DMA,
            pltpu.SemaphoreType.DMA,
            pltpu.SemaphoreType.DMA,
            pltpu.SemaphoreType.DMA,
            pltpu.VMEM_SHARED((n, d), jnp.float32),
        ],
    )
    def ek(h_hbm, src_hbm, dst_hbm, w_hbm, z_hbm, out_hbm,
           dst_all, src_r, w_r, rows, ga, gb, sa, sb, ia, ib, acc_sh):
        gsem = (ga, gb)
        ssem = (sa, sb)
        isem = (ia, ib)
        cid = lax.axis_index("c")
        sid = lax.axis_index("s")
        wid = cid * _NS + sid
        ebase = wid * per_w

        def idx_fetch(slot, c):
            pltpu.async_copy(src_hbm.at[pl.ds(ebase + c * _CH, _CH)],
                             src_r.at[slot], isem[slot % 2])
            pltpu.async_copy(w_hbm.at[pl.ds(ebase + c * _CH, _CH)],
                             w_r.at[pl.ds(slot * _CH, _CH)], isem[slot % 2])

        def idx_wait(slot):
            pltpu.make_async_copy(src_hbm.at[pl.ds(0, _CH)],
                                  src_r.at[slot],
                                  isem[slot % 2]).wait()
            pltpu.make_async_copy(w_hbm.at[pl.ds(0, _CH)],
                                  w_r.at[pl.ds(slot * _CH, _CH)],
                                  isem[slot % 2]).wait()

        def gather(buf, slot):
            pltpu.async_copy(h_hbm.at[src_r.at[slot]], rows.at[buf],
                             gsem[buf])

        def gather_wait(buf, slot):
            pltpu.make_async_copy(h_hbm.at[src_r.at[slot]], rows.at[buf],
                                  gsem[buf]).wait()

        pltpu.sync_copy(dst_hbm.at[pl.ds(wid * n_chunks, n_chunks)], dst_all)

        base = sid * slab

        @pl.when(sid < _NS - 1)
        def _():
            pltpu.sync_copy(z_hbm.at[pl.ds(base, slab)],
                            acc_sh.at[pl.ds(base, slab)])

        @pl.when(sid == _NS - 1)
        def _():
            pltpu.sync_copy(z_hbm.at[pl.ds((_NS - 1) * slab, last_slab)],
                            acc_sh.at[pl.ds((_NS - 1) * slab, last_slab)])

        plsc.subcore_barrier()

        idx_fetch(0, 0)
        idx_fetch(1, 1)
        idx_wait(0)
        gather(0, 0)

        @pl.loop(0, n_chunks, step=4)
        def _(g):
            for b in range(4):
                c = g + b
                buf = b % 2
                nbuf = (b + 1) % 2
                gather_wait(buf, b)

                @pl.when(c >= 1)
                def _():
                    pltpu.make_async_copy(
                        rows.at[nbuf], acc_sh.at[dst_all.at[c]],
                        ssem[nbuf]).wait()

                @pl.when(c + 1 < n_chunks)
                def _():
                    idx_wait((b + 1) % 4)
                    gather(nbuf, (b + 1) % 4)

                @pl.loop(0, _CH)
                def _(i):
                    idx = jnp.full((16,), b * _CH + i, jnp.int32)
                    wgt = plsc.load_gather(w_r, [idx])
                    for j in range(d // 16):
                        sl = (buf, i, pl.ds(16 * j, 16))
                        rows[sl] = rows[sl] * wgt

                pltpu.async_copy(rows.at[buf], acc_sh.at[dst_all.at[c]],
                                 ssem[buf], add=True)

                @pl.when(c + 2 < n_chunks)
                def _():
                    idx_fetch((b + 2) % 4, c + 2)

        pltpu.make_async_copy(rows.at[(n_chunks - 1) % 2],
                              acc_sh.at[dst_all.at[0]],
                              ssem[(n_chunks - 1) % 2]).wait()

        plsc.subcore_barrier()

        @pl.when(sid < _NS - 1)
        def _():
            pltpu.sync_copy(acc_sh.at[pl.ds(base, slab)],
                            out_hbm.at[cid, pl.ds(base, slab)])

        @pl.when(sid == _NS - 1)
        def _():
            pltpu.sync_copy(acc_sh.at[pl.ds((_NS - 1) * slab, last_slab)],
                            out_hbm.at[cid, pl.ds((_NS - 1) * slab, last_slab)])

    return ek(h, src_flat, dst2d, w_flat, zeros)


def _gcn_layer(x, src, dst, w, zeros, W, b, g, bt):
    h = _linear_relu(x, W, b)
    acc = _edge_pass(h, src, dst, w, zeros)
    return _combine_ln(h, acc[0], acc[1], g, bt)


def kernel(x, edge_index, edge_weights, W1, b1, g1, bt1, W2, b2, g2, bt2):
    n, d = x.shape
    src = edge_index[0].astype(jnp.int32)
    dst = edge_index[1].astype(jnp.int32)
    w = edge_weights.astype(jnp.float32)

    e = src.shape[0]
    unit = _NC * _NS * _CH * 8
    e_pad = ((e + unit - 1) // unit) * unit
    pad = e_pad - e
    if pad:
        src = jnp.concatenate([src, jnp.zeros((pad,), jnp.int32)])
        dst = jnp.concatenate([dst, jnp.zeros((pad,), jnp.int32)])
        w = jnp.concatenate([w, jnp.zeros((pad,), jnp.float32)])
    dst = dst.reshape(e_pad // _CH, _CH)
    zeros = jnp.zeros((n, d), jnp.float32)

    h = _gcn_layer(x, src, dst, w, zeros, W1, b1, g1, bt1)
    h = _gcn_layer(h, src, dst, w, zeros, W2, b2, g2, bt2)
    return h


# speedup vs baseline: 1.0670x; 1.0670x over previous
import dataclasses
import functools

import jax
import jax.numpy as jnp
from jax import lax
from jax.experimental import pallas as pl
from jax.experimental.pallas import tpu as pltpu
from jax.experimental.pallas import tpu_sc as plsc

_NC = 2
_NS = 16
_CH = 128
_NBUF = 3


def _linear_relu(x, W, b):
    n, d = x.shape
    blk = 1000

    def body(x_ref, w_ref, b_ref, o_ref):
        h = jnp.dot(x_ref[...], w_ref[...],
                    preferred_element_type=jnp.float32,
                    precision=jax.lax.Precision.HIGHEST)
        o_ref[...] = jnp.maximum(h + b_ref[...], 0.0)

    return pl.pallas_call(
        body,
        grid=(n // blk,),
        in_specs=[
            pl.BlockSpec((blk, d), lambda i: (i, 0)),
            pl.BlockSpec((d, d), lambda i: (0, 0)),
            pl.BlockSpec((1, d), lambda i: (0, 0)),
        ],
        out_specs=pl.BlockSpec((blk, d), lambda i: (i, 0)),
        out_shape=jax.ShapeDtypeStruct((n, d), jnp.float32),
    )(x, W, b.reshape(1, d))


def _combine_ln(h, a0, a1, g, bt, W=None, b=None):
    n, d = h.shape
    blk = 1000
    fused = W is not None

    def body(h_ref, a0_ref, a1_ref, g_ref, bt_ref, *rest):
        if fused:
            w_ref, b_ref, o_ref = rest
        else:
            (o_ref,) = rest
        s = h_ref[...] + a0_ref[...] + a1_ref[...]
        mu = jnp.mean(s, axis=-1, keepdims=True)
        var = jnp.mean((s - mu) ** 2, axis=-1, keepdims=True)
        t = (s - mu) * jax.lax.rsqrt(var + 1e-5) * g_ref[...] + bt_ref[...]
        if fused:
            t = jnp.dot(t, w_ref[...], preferred_element_type=jnp.float32,
                        precision=jax.lax.Precision.HIGHEST)
            t = jnp.maximum(t + b_ref[...], 0.0)
        o_ref[...] = t

    in_specs = [
        pl.BlockSpec((blk, d), lambda i: (i, 0)),
        pl.BlockSpec((blk, d), lambda i: (i, 0)),
        pl.BlockSpec((blk, d), lambda i: (i, 0)),
        pl.BlockSpec((1, d), lambda i: (0, 0)),
        pl.BlockSpec((1, d), lambda i: (0, 0)),
    ]
    args = [h, a0, a1, g.reshape(1, d), bt.reshape(1, d)]
    if fused:
        in_specs += [pl.BlockSpec((d, d), lambda i: (0, 0)),
                     pl.BlockSpec((1, d), lambda i: (0, 0))]
        args += [W, b.reshape(1, d)]

    return pl.pallas_call(
        body,
        grid=(n // blk,),
        in_specs=in_specs,
        out_specs=pl.BlockSpec((blk, d), lambda i: (i, 0)),
        out_shape=jax.ShapeDtypeStruct((n, d), jnp.float32),
    )(*args)


def _edge_pass(h, src_flat, dst2d, w_flat, zeros):
    n, d = h.shape
    nw = _NC * _NS
    per_w = src_flat.shape[0] // nw
    n_chunks = per_w // _CH
    assert n_chunks % 4 == 0 and n_chunks >= 8
    slab = ((n + _NS - 1) // _NS + 7) // 8 * 8
    last_slab = n - (_NS - 1) * slab
    assert last_slab > 0 and last_slab % 8 == 0

    mesh = plsc.VectorSubcoreMesh(core_axis_name="c", subcore_axis_name="s")

    cp = pltpu.CompilerParams()
    if "needs_layout_passes" in pltpu.CompilerParams.__dataclass_fields__:
        cp = dataclasses.replace(cp, needs_layout_passes=False)

    @functools.partial(
        pl.kernel,
        compiler_params=cp,
        out_type=jax.ShapeDtypeStruct((_NC, n, d), jnp.float32),
        mesh=mesh,
        scratch_types=[
            pltpu.VMEM((n_chunks, _CH), jnp.int32),
            pltpu.VMEM((4, _CH), jnp.int32),
            pltpu.VMEM((4 * _CH,), jnp.float32),
            pltpu.VMEM((2, _CH, d), jnp.float32),
            pltpu.SemaphoreType.DMA,
            pltpu.SemaphoreType.DMA,
            pltpu.SemaphoreType.DMA,
            pltpu.SemaphoreType.DMA,
            pltpu.SemaphoreType.DMA,
            pltpu.SemaphoreType.DMA,
            pltpu.VMEM_SHARED((n, d), jnp.float32),
        ],
    )
    def ek(h_hbm, src_hbm, dst_hbm, w_hbm, z_hbm, out_hbm,
           dst_all, src_r, w_r, rows, ga, gb, sa, sb, ia, ib, acc_sh):
        gsem = (ga, gb)
        ssem = (sa, sb)
        isem = (ia, ib)
        cid = lax.axis_index("c")
        sid = lax.axis_index("s")
        wid = cid * _NS + sid
        ebase = wid * per_w

        def idx_fetch(slot, c):
            pltpu.async_copy(src_hbm.at[pl.ds(ebase + c * _CH, _CH)],
                             src_r.at[slot], isem[slot % 2])
            pltpu.async_copy(w_hbm.at[pl.ds(ebase + c * _CH, _CH)],
                             w_r.at[pl.ds(slot * _CH, _CH)], isem[slot % 2])

        def idx_wait(slot):
            pltpu.make_async_copy(src_hbm.at[pl.ds(0, _CH)],
                                  src_r.at[slot],
                                  isem[slot % 2]).wait()
            pltpu.make_async_copy(w_hbm.at[pl.ds(0, _CH)],
                                  w_r.at[pl.ds(slot * _CH, _CH)],
                                  isem[slot % 2]).wait()

        def gather(buf, slot):
            pltpu.async_copy(h_hbm.at[src_r.at[slot]], rows.at[buf],
                             gsem[buf])

        def gather_wait(buf, slot):
            pltpu.make_async_copy(h_hbm.at[src_r.at[slot]], rows.at[buf],
                                  gsem[buf]).wait()

        pltpu.sync_copy(dst_hbm.at[pl.ds(wid * n_chunks, n_chunks)], dst_all)

        base = sid * slab

        @pl.when(sid < _NS - 1)
        def _():
            pltpu.sync_copy(z_hbm.at[pl.ds(base, slab)],
                            acc_sh.at[pl.ds(base, slab)])

        @pl.when(sid == _NS - 1)
        def _():
            pltpu.sync_copy(z_hbm.at[pl.ds((_NS - 1) * slab, last_slab)],
                            acc_sh.at[pl.ds((_NS - 1) * slab, last_slab)])

        plsc.subcore_barrier()

        idx_fetch(0, 0)
        idx_fetch(1, 1)
        idx_wait(0)
        gather(0, 0)

        @pl.loop(0, n_chunks, step=4)
        def _(g):
            for b in range(4):
                c = g + b
                buf = b % 2
                nbuf = (b + 1) % 2
                gather_wait(buf, b)

                @pl.when(c >= 1)
                def _():
                    pltpu.make_async_copy(
                        rows.at[nbuf], acc_sh.at[dst_all.at[c]],
                        ssem[nbuf]).wait()

                @pl.when(c + 1 < n_chunks)
                def _():
                    idx_wait((b + 1) % 4)
                    gather(nbuf, (b + 1) % 4)

                @pl.loop(0, _CH)
                def _(i):
                    idx = jnp.full((16,), b * _CH + i, jnp.int32)
                    wgt = plsc.load_gather(w_r, [idx])
                    for j in range(d // 16):
                        sl = (buf, i, pl.ds(16 * j, 16))
                        rows[sl] = rows[sl] * wgt

                pltpu.async_copy(rows.at[buf], acc_sh.at[dst_all.at[c]],
                                 ssem[buf], add=True)

                @pl.when(c + 2 < n_chunks)
                def _():
                    idx_fetch((b + 2) % 4, c + 2)

        pltpu.make_async_copy(rows.at[(n_chunks - 1) % 2],
                              acc_sh.at[dst_all.at[0]],
                              ssem[(n_chunks - 1) % 2]).wait()

        plsc.subcore_barrier()

        @pl.when(sid < _NS - 1)
        def _():
            pltpu.sync_copy(acc_sh.at[pl.ds(base, slab)],
                            out_hbm.at[cid, pl.ds(base, slab)])

        @pl.when(sid == _NS - 1)
        def _():
            pltpu.sync_copy(acc_sh.at[pl.ds((_NS - 1) * slab, last_slab)],
                            out_hbm.at[cid, pl.ds((_NS - 1) * slab, last_slab)])

    return ek(h, src_flat, dst2d, w_flat, zeros)


def kernel(x, edge_index, edge_weights, W1, b1, g1, bt1, W2, b2, g2, bt2):
    n, d = x.shape
    src = edge_index[0].astype(jnp.int32)
    dst = edge_index[1].astype(jnp.int32)
    w = edge_weights.astype(jnp.float32)

    e = src.shape[0]
    unit = _NC * _NS * _CH * 8
    e_pad = ((e + unit - 1) // unit) * unit
    pad = e_pad - e
    if pad:
        src = jnp.concatenate([src, jnp.zeros((pad,), jnp.int32)])
        dst = jnp.concatenate([dst, jnp.zeros((pad,), jnp.int32)])
        w = jnp.concatenate([w, jnp.zeros((pad,), jnp.float32)])
    dst = dst.reshape(e_pad // _CH, _CH)
    zeros = jnp.zeros((n, d), jnp.float32)

    h1 = _linear_relu(x, W1, b1)
    acc = _edge_pass(h1, src, dst, w, zeros)
    h2 = _combine_ln(h1, acc[0], acc[1], g1, bt1, W2, b2)
    acc = _edge_pass(h2, src, dst, w, zeros)
    return _combine_ln(h2, acc[0], acc[1], g2, bt2)
